# baseline (speedup 1.0000x reference)
import jax
import jax.numpy as jnp
from jax.experimental import pallas as pl
from jax.experimental.pallas import tpu as pltpu

TOK = 1024
EP = 128


def _gating_body(x_ref, w_ref, b_ref, o_ref):
    xh = x_ref[...].astype(jnp.bfloat16)
    wh = w_ref[...].astype(jnp.bfloat16)
    logits = jnp.dot(xh, wh, preferred_element_type=jnp.float32)
    logits = logits + b_ref[...][None, :]
    m = jnp.max(logits, axis=-1, keepdims=True)
    e = jnp.exp(logits - m)
    o_ref[...] = e / jnp.sum(e, axis=-1, keepdims=True)


def kernel(x, W, b):
    B, S, D = x.shape
    E = W.shape[1]
    N = B * S
    xf = x.reshape(N, D)
    Wp = jnp.pad(W, ((0, 0), (0, EP - E)))
    bp = jnp.pad(b, (0, EP - E), constant_values=-jnp.inf)

    out = pl.pallas_call(
        _gating_body,
        grid=(N // TOK,),
        in_specs=[
            pl.BlockSpec((TOK, D), lambda i: (i, 0)),
            pl.BlockSpec((D, EP), lambda i: (0, 0)),
            pl.BlockSpec((EP,), lambda i: (0,)),
        ],
        out_specs=pl.BlockSpec((TOK, EP), lambda i: (i, 0)),
        out_shape=jax.ShapeDtypeStruct((N, EP), jnp.float32),
    )(xf, Wp, bp)
    return out[:, :E].reshape(B, S, E)


# speedup vs baseline: 1.0745x; 1.0745x over previous
import jax
import jax.numpy as jnp
from jax.experimental import pallas as pl
from jax.experimental.pallas import tpu as pltpu

TOK = 1024
EP = 128


def _gating_body(x_ref, w_ref, b_ref, o_ref):
    xh = x_ref[...].astype(jnp.bfloat16)
    wh = w_ref[...].astype(jnp.bfloat16)
    logits = jnp.dot(xh, wh, preferred_element_type=jnp.float32)
    logits = logits + b_ref[...][None, :]
    m = jnp.max(logits, axis=-1, keepdims=True)
    e = jnp.exp(logits - m)
    p = e / jnp.sum(e, axis=-1, keepdims=True)
    o_ref[...] = p[None, :, :o_ref.shape[2]]


def kernel(x, W, b):
    B, S, D = x.shape
    E = W.shape[1]
    N = B * S
    xf = x.reshape(N, D)
    Wp = jnp.pad(W, ((0, 0), (0, EP - E)))
    bp = jnp.pad(b, (0, EP - E), constant_values=-jnp.inf)

    out = pl.pallas_call(
        _gating_body,
        grid=(N // TOK,),
        in_specs=[
            pl.BlockSpec((TOK, D), lambda i: (i, 0)),
            pl.BlockSpec((D, EP), lambda i: (0, 0)),
            pl.BlockSpec((EP,), lambda i: (0,)),
        ],
        out_specs=pl.BlockSpec((1, TOK, E),
                               lambda i: (i // (S // TOK), i % (S // TOK), 0)),
        out_shape=jax.ShapeDtypeStruct((B, S, E), jnp.float32),
    )(xf, Wp, bp)
    return out


# speedup vs baseline: 1.0826x; 1.0075x over previous
import jax
import jax.numpy as jnp
from jax.experimental import pallas as pl
from jax.experimental.pallas import tpu as pltpu

TOK = 1024
EP = 128


def _gating_body(x_ref, w_ref, b_ref, o_ref):
    xh = x_ref[...].astype(jnp.bfloat16)
    wh = w_ref[...].astype(jnp.bfloat16)
    logits = jnp.dot(xh, wh, preferred_element_type=jnp.float32)
    logits = logits + b_ref[...][None, :]
    m = jnp.max(logits, axis=-1, keepdims=True)
    e = jnp.exp(logits - m)
    p = e / jnp.sum(e, axis=-1, keepdims=True)
    o_ref[...] = p[None, :, :o_ref.shape[2]]


def kernel(x, W, b):
    B, S, D = x.shape
    E = W.shape[1]
    N = B * S
    xf = x.reshape(N, D)


    out = pl.pallas_call(
        _gating_body,
        grid=(N // TOK,),
        in_specs=[
            pl.BlockSpec((TOK, D), lambda i: (i, 0)),
            pl.BlockSpec((D, E), lambda i: (0, 0)),
            pl.BlockSpec((E,), lambda i: (0,)),
        ],
        out_specs=pl.BlockSpec((1, TOK, E),
                               lambda i: (i // (S // TOK), i % (S // TOK), 0)),
        out_shape=jax.ShapeDtypeStruct((B, S, E), jnp.float32),
    )(xf, W, b)
    return out
